# baseline (speedup 1.0000x reference)
import jax
import jax.numpy as jnp
from jax import lax
from jax.experimental import pallas as pl
from jax.experimental.pallas import tpu as pltpu
from jax.experimental.pallas import tpu_sc as plsc

K = 8
S = 4
B, C, T = 8, 768, 4096
R = B * C
NCHUNK = T // S
TP = (T - K) // S + 1
NW = 32
RPW = R // NW
G = 8
NG = RPW // G
GI = G * T
GO = G * TP
OB = GO + 8


def _pool_body(x_hbm, o_hbm, xbuf, obuf, abuf, qbuf, si0, si1, so0, so1):
    cid = lax.axis_index("c")
    sid = lax.axis_index("s")
    wid = sid * 2 + cid
    row0 = wid * RPW
    lane4 = lax.iota(jnp.int32, 16) * 4
    lane4p1 = lane4 + 1
    lane4p2 = lane4 + 2
    lane4p3 = lane4 + 3

    def in_copy(g, slot, sem):
        base = row0 + g * G
        return pltpu.make_async_copy(
            x_hbm.at[pl.ds(base * T, GI)],
            xbuf.at[pl.ds(slot * GI, GI)],
            sem,
        )

    def out_copy(g, slot, sem):
        base = row0 + g * G
        return pltpu.make_async_copy(
            obuf.at[pl.ds(slot * OB, GO)],
            o_hbm.at[pl.ds(base * TP, GO)],
            sem,
        )

    def compute_group(slot):
        xb = slot * GI
        ob = slot * OB

        def do_row(r, _):
            roff = xb + r * T

            def pass1(i, _):
                src = xbuf.at[pl.ds(roff + 64 * i, 64)]
                g0 = plsc.load_gather(src, [lane4])
                g1 = plsc.load_gather(src, [lane4p1])
                g2 = plsc.load_gather(src, [lane4p2])
                g3 = plsc.load_gather(src, [lane4p3])
                abuf[pl.ds(i * 16, 16)] = (g0 + g1) + (g2 + g3)
                qbuf[pl.ds(i * 16, 16)] = (g0 * g0 + g1 * g1) + (
                    g2 * g2 + g3 * g3
                )
                return 0

            lax.fori_loop(0, NCHUNK // 16, pass1, 0, unroll=4)
            ooff = ob + r * TP

            def pass2(i, _):
                t = i * 16
                a_lo = abuf[pl.ds(t, 16)]
                a_hi = abuf[pl.ds(t + 1, 16)]
                q_lo = qbuf[pl.ds(t, 16)]
                q_hi = qbuf[pl.ds(t + 1, 16)]
                den = a_lo + a_hi
                num = q_lo + q_hi
                r = plsc.bitcast(
                    jnp.int32(0x7EF311C5) - plsc.bitcast(den, jnp.int32),
                    jnp.float32,
                )
                r = r * (2.0 - den * r)
                r = r * (2.0 - den * r)
                res = jnp.where(den > 0.0, num * r, 0.0)
                obuf[pl.ds(ooff + t, 16)] = res
                return 0

            lax.fori_loop(0, NCHUNK // 16, pass2, 0, unroll=4)
            return 0

        lax.fori_loop(0, G, do_row, 0)

    in_copy(0, 0, si0).start()
    in_copy(1, 1, si1).start()

    def do_pair(j, _):
        g = 2 * j

        in_copy(g, 0, si0).wait()

        @pl.when(j > 0)
        def _():
            out_copy(g - 2, 0, so0).wait()

        compute_group(0)
        out_copy(g, 0, so0).start()

        @pl.when(j < NG // 2 - 1)
        def _():
            in_copy(g + 2, 0, si0).start()

        in_copy(g + 1, 1, si1).wait()

        @pl.when(j > 0)
        def _():
            out_copy(g - 1, 1, so1).wait()

        compute_group(1)
        out_copy(g + 1, 1, so1).start()

        @pl.when(j < NG // 2 - 1)
        def _():
            in_copy(g + 3, 1, si1).start()

        return 0

    lax.fori_loop(0, NG // 2, do_pair, 0)
    out_copy(NG - 2, 0, so0).wait()
    out_copy(NG - 1, 1, so1).wait()


def kernel(input):
    x = input.reshape(R * T)
    mesh = plsc.VectorSubcoreMesh(
        core_axis_name="c", subcore_axis_name="s", num_cores=2, num_subcores=16
    )
    run = pl.kernel(
        _pool_body,
        out_type=jax.ShapeDtypeStruct((R * TP,), jnp.float32),
        mesh=mesh,
        compiler_params=pltpu.CompilerParams(needs_layout_passes=False),
        scratch_types=[
            pltpu.VMEM((2 * GI,), jnp.float32),
            pltpu.VMEM((2 * OB,), jnp.float32),
            pltpu.VMEM((NCHUNK + 16,), jnp.float32),
            pltpu.VMEM((NCHUNK + 16,), jnp.float32),
            pltpu.SemaphoreType.DMA,
            pltpu.SemaphoreType.DMA,
            pltpu.SemaphoreType.DMA,
            pltpu.SemaphoreType.DMA,
        ],
    )
    out = run(x)
    return out.reshape(B, C, TP)


# speedup vs baseline: 1.0517x; 1.0517x over previous
import jax
import jax.numpy as jnp
from jax import lax
from jax.experimental import pallas as pl
from jax.experimental.pallas import tpu as pltpu
from jax.experimental.pallas import tpu_sc as plsc

K = 8
S = 4
B, C, T = 8, 768, 4096
R = B * C
NCHUNK = T // S
TP = (T - K) // S + 1
NW = 32
RPW = R // NW
G = 8
NG = RPW // G
GI = G * T
GO = G * TP
OB = GO + 8


def _pool_body(x_hbm, o_hbm, xbuf, obuf, abuf, qbuf, si0, si1, so0, so1):
    cid = lax.axis_index("c")
    sid = lax.axis_index("s")
    wid = sid * 2 + cid
    row0 = wid * RPW
    lane4 = lax.iota(jnp.int32, 16) * 4
    lane4p1 = lane4 + 1
    lane4p2 = lane4 + 2
    lane4p3 = lane4 + 3

    def in_copy(g, slot, sem):
        base = row0 + g * G
        return pltpu.make_async_copy(
            x_hbm.at[pl.ds(base * T, GI)],
            xbuf.at[pl.ds(slot * GI, GI)],
            sem,
        )

    def out_copy(g, slot, sem):
        base = row0 + g * G
        return pltpu.make_async_copy(
            obuf.at[pl.ds(slot * OB, GO)],
            o_hbm.at[pl.ds(base * TP, GO)],
            sem,
        )

    def compute_group(slot):
        xb = slot * GI
        ob = slot * OB

        def do_row(r, _):
            roff = xb + r * T

            def pass1(i, _):
                src = xbuf.at[pl.ds(roff + 64 * i, 64)]
                g0 = plsc.load_gather(src, [lane4])
                g1 = plsc.load_gather(src, [lane4p1])
                g2 = plsc.load_gather(src, [lane4p2])
                g3 = plsc.load_gather(src, [lane4p3])
                abuf[pl.ds(i * 16, 16)] = (g0 + g1) + (g2 + g3)
                qbuf[pl.ds(i * 16, 16)] = (g0 * g0 + g1 * g1) + (
                    g2 * g2 + g3 * g3
                )
                return 0

            lax.fori_loop(0, NCHUNK // 16, pass1, 0, unroll=4)
            ooff = ob + r * TP

            def pass2(i, _):
                t = i * 16
                a_lo = abuf[pl.ds(t, 16)]
                a_hi = abuf[pl.ds(t + 1, 16)]
                q_lo = qbuf[pl.ds(t, 16)]
                q_hi = qbuf[pl.ds(t + 1, 16)]
                den = a_lo + a_hi
                num = q_lo + q_hi
                res = jnp.where(den > 0.0, num / den, 0.0)
                obuf[pl.ds(ooff + t, 16)] = res
                return 0

            lax.fori_loop(0, NCHUNK // 16, pass2, 0, unroll=4)
            return 0

        lax.fori_loop(0, G, do_row, 0)

    in_copy(0, 0, si0).start()
    in_copy(1, 1, si1).start()

    def do_pair(j, _):
        g = 2 * j

        in_copy(g, 0, si0).wait()

        @pl.when(j > 0)
        def _():
            out_copy(g - 2, 0, so0).wait()

        compute_group(0)
        out_copy(g, 0, so0).start()

        @pl.when(j < NG // 2 - 1)
        def _():
            in_copy(g + 2, 0, si0).start()

        in_copy(g + 1, 1, si1).wait()

        @pl.when(j > 0)
        def _():
            out_copy(g - 1, 1, so1).wait()

        compute_group(1)
        out_copy(g + 1, 1, so1).start()

        @pl.when(j < NG // 2 - 1)
        def _():
            in_copy(g + 3, 1, si1).start()

        return 0

    lax.fori_loop(0, NG // 2, do_pair, 0)
    out_copy(NG - 2, 0, so0).wait()
    out_copy(NG - 1, 1, so1).wait()


def kernel(input):
    x = input.reshape(R * T)
    mesh = plsc.VectorSubcoreMesh(
        core_axis_name="c", subcore_axis_name="s", num_cores=2, num_subcores=16
    )
    run = pl.kernel(
        _pool_body,
        out_type=jax.ShapeDtypeStruct((R * TP,), jnp.float32),
        mesh=mesh,
        compiler_params=pltpu.CompilerParams(needs_layout_passes=False),
        scratch_types=[
            pltpu.VMEM((2 * GI,), jnp.float32),
            pltpu.VMEM((2 * OB,), jnp.float32),
            pltpu.VMEM((NCHUNK + 16,), jnp.float32),
            pltpu.VMEM((NCHUNK + 16,), jnp.float32),
            pltpu.SemaphoreType.DMA,
            pltpu.SemaphoreType.DMA,
            pltpu.SemaphoreType.DMA,
            pltpu.SemaphoreType.DMA,
        ],
    )
    out = run(x)
    return out.reshape(B, C, TP)


# speedup vs baseline: 1.2723x; 1.2097x over previous
import jax
import jax.numpy as jnp
from jax import lax
from jax.experimental import pallas as pl
from jax.experimental.pallas import tpu as pltpu
from jax.experimental.pallas import tpu_sc as plsc

K = 8
S = 4
B, C, T = 8, 768, 4096
R = B * C
NCHUNK = T // S
TP = (T - K) // S + 1
NW = 32
RPW = R // NW
G = 8
NG = RPW // G
GO = G * TP
OB = GO + 8


def _pool_body(x_hbm, o_hbm, xbuf, obuf, abuf, qbuf, si0, si1, so0, so1):
    cid = lax.axis_index("c")
    sid = lax.axis_index("s")
    wid = sid * 2 + cid
    row0 = wid * RPW
    lane4 = lax.iota(jnp.int32, 16) * 4
    lane4p1 = lane4 + 1
    lane4p2 = lane4 + 2
    lane4p3 = lane4 + 3
    zero16 = jnp.zeros((16,), jnp.int32)

    def in_copy(g, slot, sem):
        base = row0 + g * G
        return pltpu.make_async_copy(
            x_hbm.at[pl.ds(base, G)],
            xbuf.at[slot],
            sem,
        )

    def out_copy(g, slot, sem):
        base = row0 + g * G
        return pltpu.make_async_copy(
            obuf.at[pl.ds(slot * OB, GO)],
            o_hbm.at[pl.ds(base * TP, GO)],
            sem,
        )

    def compute_group(slot):
        ob = slot * OB
        xslot = xbuf.at[slot]

        def do_row(r, _):
            rvec = zero16 + r

            def pass1(i, _):
                c0 = 64 * i
                g0 = plsc.load_gather(xslot, [rvec, lane4 + c0])
                g1 = plsc.load_gather(xslot, [rvec, lane4p1 + c0])
                g2 = plsc.load_gather(xslot, [rvec, lane4p2 + c0])
                g3 = plsc.load_gather(xslot, [rvec, lane4p3 + c0])
                abuf[pl.ds(i * 16, 16)] = (g0 + g1) + (g2 + g3)
                qbuf[pl.ds(i * 16, 16)] = (g0 * g0 + g1 * g1) + (
                    g2 * g2 + g3 * g3
                )
                return 0

            lax.fori_loop(0, NCHUNK // 16, pass1, 0, unroll=4)
            ooff = ob + r * TP

            def pass2(i, _):
                t = i * 16
                a_lo = abuf[pl.ds(t, 16)]
                a_hi = abuf[pl.ds(t + 1, 16)]
                q_lo = qbuf[pl.ds(t, 16)]
                q_hi = qbuf[pl.ds(t + 1, 16)]
                den = a_lo + a_hi
                num = q_lo + q_hi
                res = jnp.where(den > 0.0, num / den, 0.0)
                obuf[pl.ds(ooff + t, 16)] = res
                return 0

            lax.fori_loop(0, NCHUNK // 16, pass2, 0, unroll=4)
            return 0

        lax.fori_loop(0, G, do_row, 0)

    in_copy(0, 0, si0).start()
    in_copy(1, 1, si1).start()

    def do_pair(j, _):
        g = 2 * j

        in_copy(g, 0, si0).wait()

        @pl.when(j > 0)
        def _():
            out_copy(g - 2, 0, so0).wait()

        compute_group(0)
        out_copy(g, 0, so0).start()

        @pl.when(j < NG // 2 - 1)
        def _():
            in_copy(g + 2, 0, si0).start()

        in_copy(g + 1, 1, si1).wait()

        @pl.when(j > 0)
        def _():
            out_copy(g - 1, 1, so1).wait()

        compute_group(1)
        out_copy(g + 1, 1, so1).start()

        @pl.when(j < NG // 2 - 1)
        def _():
            in_copy(g + 3, 1, si1).start()

        return 0

    lax.fori_loop(0, NG // 2, do_pair, 0)
    out_copy(NG - 2, 0, so0).wait()
    out_copy(NG - 1, 1, so1).wait()


def kernel(input):
    x = input.reshape(R, T)
    mesh = plsc.VectorSubcoreMesh(
        core_axis_name="c", subcore_axis_name="s", num_cores=2, num_subcores=16
    )
    run = pl.kernel(
        _pool_body,
        out_type=jax.ShapeDtypeStruct((R * TP,), jnp.float32),
        mesh=mesh,
        compiler_params=pltpu.CompilerParams(needs_layout_passes=False),
        scratch_types=[
            pltpu.VMEM((2, G, T), jnp.float32),
            pltpu.VMEM((2 * OB,), jnp.float32),
            pltpu.VMEM((NCHUNK + 16,), jnp.float32),
            pltpu.VMEM((NCHUNK + 16,), jnp.float32),
            pltpu.SemaphoreType.DMA,
            pltpu.SemaphoreType.DMA,
            pltpu.SemaphoreType.DMA,
            pltpu.SemaphoreType.DMA,
        ],
    )
    out = run(x)
    return out.reshape(B, C, TP)


# speedup vs baseline: 2.9595x; 2.3261x over previous
import jax
import jax.numpy as jnp
from jax import lax
from jax.experimental import pallas as pl
from jax.experimental.pallas import tpu as pltpu
from jax.experimental.pallas import tpu_sc as plsc

K = 8
S = 4
B, C, T = 8, 768, 4096
R = B * C
NCHUNK = T // S
TP = (T - K) // S + 1
NW = 32
RPW = R // NW
G = 8
NG = RPW // G
GO = G * TP
OB = GO + 8


def _pool_body(x_hbm, o_hbm, xbuf, obuf, abuf, qbuf, si0, si1, so0, so1):
    cid = lax.axis_index("c")
    sid = lax.axis_index("s")
    wid = sid * 2 + cid
    row0 = wid * RPW
    lane4 = lax.iota(jnp.int32, 16) * 4
    lane4p1 = lane4 + 1
    lane4p2 = lane4 + 2
    lane4p3 = lane4 + 3
    zero16 = jnp.zeros((16,), jnp.int32)

    def in_copy(g, slot, sem):
        base = row0 + g * G
        return pltpu.make_async_copy(
            x_hbm.at[pl.ds(base, G)],
            xbuf.at[slot],
            sem,
        )

    def out_copy(g, slot, sem):
        base = row0 + g * G
        return pltpu.make_async_copy(
            obuf.at[pl.ds(slot * OB, GO)],
            o_hbm.at[pl.ds(base * TP, GO)],
            sem,
        )

    def compute_group(slot):
        ob = slot * OB
        xslot = xbuf.at[slot]

        def do_row(r, _):
            rvec = zero16 + r

            def pass1(i):
                c0 = 64 * i
                g0 = plsc.load_gather(xslot, [rvec, lane4 + c0])
                g1 = plsc.load_gather(xslot, [rvec, lane4p1 + c0])
                g2 = plsc.load_gather(xslot, [rvec, lane4p2 + c0])
                g3 = plsc.load_gather(xslot, [rvec, lane4p3 + c0])
                abuf[pl.ds(i * 16, 16)] = (g0 + g1) + (g2 + g3)
                qbuf[pl.ds(i * 16, 16)] = (g0 * g0 + g1 * g1) + (
                    g2 * g2 + g3 * g3
                )

            plsc.parallel_loop(0, NCHUNK // 16, unroll=4)(pass1)
            ooff = ob + r * TP

            def pass2(i):
                t = i * 16
                a_lo = abuf[pl.ds(t, 16)]
                a_hi = abuf[pl.ds(t + 1, 16)]
                q_lo = qbuf[pl.ds(t, 16)]
                q_hi = qbuf[pl.ds(t + 1, 16)]
                den = a_lo + a_hi
                num = q_lo + q_hi
                res = jnp.where(den > 0.0, num / den, 0.0)
                obuf[pl.ds(ooff + t, 16)] = res

            plsc.parallel_loop(0, NCHUNK // 16, unroll=4)(pass2)
            return 0

        lax.fori_loop(0, G, do_row, 0)

    in_copy(0, 0, si0).start()
    in_copy(1, 1, si1).start()

    def do_pair(j, _):
        g = 2 * j

        in_copy(g, 0, si0).wait()

        @pl.when(j > 0)
        def _():
            out_copy(g - 2, 0, so0).wait()

        compute_group(0)
        out_copy(g, 0, so0).start()

        @pl.when(j < NG // 2 - 1)
        def _():
            in_copy(g + 2, 0, si0).start()

        in_copy(g + 1, 1, si1).wait()

        @pl.when(j > 0)
        def _():
            out_copy(g - 1, 1, so1).wait()

        compute_group(1)
        out_copy(g + 1, 1, so1).start()

        @pl.when(j < NG // 2 - 1)
        def _():
            in_copy(g + 3, 1, si1).start()

        return 0

    lax.fori_loop(0, NG // 2, do_pair, 0)
    out_copy(NG - 2, 0, so0).wait()
    out_copy(NG - 1, 1, so1).wait()


def kernel(input):
    x = input.reshape(R, T)
    mesh = plsc.VectorSubcoreMesh(
        core_axis_name="c", subcore_axis_name="s", num_cores=2, num_subcores=16
    )
    run = pl.kernel(
        _pool_body,
        out_type=jax.ShapeDtypeStruct((R * TP,), jnp.float32),
        mesh=mesh,
        compiler_params=pltpu.CompilerParams(needs_layout_passes=False),
        scratch_types=[
            pltpu.VMEM((2, G, T), jnp.float32),
            pltpu.VMEM((2 * OB,), jnp.float32),
            pltpu.VMEM((NCHUNK + 16,), jnp.float32),
            pltpu.VMEM((NCHUNK + 16,), jnp.float32),
            pltpu.SemaphoreType.DMA,
            pltpu.SemaphoreType.DMA,
            pltpu.SemaphoreType.DMA,
            pltpu.SemaphoreType.DMA,
        ],
    )
    out = run(x)
    return out.reshape(B, C, TP)
